# initial kernel scaffold (unmeasured)
import jax
import jax.numpy as jnp
from jax import lax
from jax.experimental import pallas as pl
from jax.experimental.pallas import tpu as pltpu

N_DEV = 4
N_TOK = 4096
D_IN = 1024
D_OUT = 2048
E_LOCAL = 4
TOK_BLK = 512
CHUNK = N_TOK // N_DEV

_DevIdType = getattr(pl, "DeviceIdType", None) or pltpu.DeviceIdType
_CompilerParams = getattr(pltpu, "CompilerParams", None) or getattr(
    pltpu, "TPUCompilerParams"
)
_sem_signal = getattr(pl, "semaphore_signal", None) or pltpu.semaphore_signal
_sem_wait = getattr(pl, "semaphore_wait", None) or pltpu.semaphore_wait


def _compute_partial(x, route_idx, expert_W):
    n_i = N_TOK // TOK_BLK

    def body(route_ref, x_ref, w_ref, out_ref):
        my = lax.axis_index("i")
        e = pl.program_id(1)
        eid = my * E_LOCAL + e
        mask = route_ref[...] == eid
        xm = jnp.where(mask, x_ref[...], 0.0)
        acc = jnp.dot(xm, w_ref[0], preferred_element_type=jnp.float32)

        @pl.when(e == 0)
        def _():
            out_ref[...] = acc

        @pl.when(e != 0)
        def _():
            out_ref[...] += acc

    return pl.pallas_call(
        body,
        grid=(n_i, E_LOCAL),
        in_specs=[
            pl.BlockSpec((TOK_BLK, 1), lambda i, e: (i, 0)),
            pl.BlockSpec((TOK_BLK, D_IN), lambda i, e: (i, 0)),
            pl.BlockSpec((1, D_IN, D_OUT), lambda i, e: (e, 0, 0)),
        ],
        out_specs=pl.BlockSpec((TOK_BLK, D_OUT), lambda i, e: (i, 0)),
        out_shape=jax.ShapeDtypeStruct((N_TOK, D_OUT), jnp.float32),
        compiler_params=_CompilerParams(
            dimension_semantics=("arbitrary", "arbitrary"),
        ),
    )(route_idx, x, expert_W)


def _ring_allreduce(partial):

    def body(in_ref, out_ref, comm_ref, send_sems, recv_sems):
        my = lax.axis_index("i")
        left = lax.rem(my + N_DEV - 1, N_DEV)
        right = lax.rem(my + 1, N_DEV)

        barrier = pltpu.get_barrier_semaphore()
        _sem_signal(
            barrier, inc=1, device_id=(left,), device_id_type=_DevIdType.MESH
        )
        _sem_signal(
            barrier, inc=1, device_id=(right,), device_id_type=_DevIdType.MESH
        )
        _sem_wait(barrier, 2)

        for s in range(N_DEV - 1):
            sc = lax.rem(my - s + N_DEV, N_DEV)
            rc = lax.rem(my - s - 1 + N_DEV, N_DEV)
            rdma = pltpu.make_async_remote_copy(
                src_ref=out_ref.at[pl.ds(sc * CHUNK, CHUNK), :],
                dst_ref=comm_ref.at[s],
                send_sem=send_sems.at[s],
                recv_sem=recv_sems.at[s],
                device_id=(right,),
                device_id_type=_DevIdType.MESH,
            )
            rdma.start()
            rdma.wait()
            out_ref[pl.ds(rc * CHUNK, CHUNK), :] += comm_ref[s]

        for s in range(N_DEV - 1):
            sc = lax.rem(my + 1 - s + N_DEV, N_DEV)
            h = N_DEV - 1 + s
            rdma = pltpu.make_async_remote_copy(
                src_ref=out_ref.at[pl.ds(sc * CHUNK, CHUNK), :],
                dst_ref=out_ref.at[pl.ds(sc * CHUNK, CHUNK), :],
                send_sem=send_sems.at[h],
                recv_sem=recv_sems.at[h],
                device_id=(right,),
                device_id_type=_DevIdType.MESH,
            )
            rdma.start()
            rdma.wait()

    return pl.pallas_call(
        body,
        out_shape=jax.ShapeDtypeStruct((N_TOK, D_OUT), jnp.float32),
        in_specs=[pl.BlockSpec(memory_space=pltpu.VMEM)],
        out_specs=pl.BlockSpec(memory_space=pltpu.VMEM),
        scratch_shapes=[
            pltpu.VMEM((N_DEV - 1, CHUNK, D_OUT), jnp.float32),
            pltpu.SemaphoreType.DMA((2 * (N_DEV - 1),)),
            pltpu.SemaphoreType.DMA((2 * (N_DEV - 1),)),
        ],
        input_output_aliases={0: 0},
        compiler_params=_CompilerParams(collective_id=0),
    )(partial)


def kernel(x, router_W, route_idx, expert_W):
    del router_W
    partial = _compute_partial(x, route_idx, expert_W)
    return _ring_allreduce(partial)


# baseline (device time: 722956 ns/iter reference)
import jax
import jax.numpy as jnp
from jax import lax
from jax.experimental import pallas as pl
from jax.experimental.pallas import tpu as pltpu

N_DEV = 4
N_TOK = 4096
D_IN = 1024
D_OUT = 2048
E_LOCAL = 4
TOK_BLK = 512
CHUNK = N_TOK // N_DEV

_DevIdType = getattr(pl, "DeviceIdType", None) or pltpu.DeviceIdType
_CompilerParams = getattr(pltpu, "CompilerParams", None) or getattr(
    pltpu, "TPUCompilerParams"
)
_sem_signal = getattr(pl, "semaphore_signal", None) or pltpu.semaphore_signal
_sem_wait = getattr(pl, "semaphore_wait", None) or pltpu.semaphore_wait
_ANY = pl.ANY
_VMEM_SPACE = pltpu.MemorySpace.VMEM


def _compute_partial(x, route_idx, expert_W):
    n_i = N_TOK // TOK_BLK

    def body(route_ref, x_ref, w_ref, out_ref):
        my = lax.axis_index("i")
        e = pl.program_id(1)
        eid = my * E_LOCAL + e
        mask = route_ref[...] == eid
        xm = jnp.where(mask, x_ref[...], 0.0)
        acc = jnp.dot(xm, w_ref[0], preferred_element_type=jnp.float32)

        @pl.when(e == 0)
        def _():
            out_ref[...] = acc

        @pl.when(e != 0)
        def _():
            out_ref[...] += acc

    return pl.pallas_call(
        body,
        grid=(n_i, E_LOCAL),
        in_specs=[
            pl.BlockSpec((TOK_BLK, 1), lambda i, e: (i, 0)),
            pl.BlockSpec((TOK_BLK, D_IN), lambda i, e: (i, 0)),
            pl.BlockSpec((1, D_IN, D_OUT), lambda i, e: (e, 0, 0)),
        ],
        out_specs=pl.BlockSpec((TOK_BLK, D_OUT), lambda i, e: (i, 0)),
        out_shape=jax.ShapeDtypeStruct((N_TOK, D_OUT), jnp.float32),
        compiler_params=_CompilerParams(
            dimension_semantics=("arbitrary", "arbitrary"),
        ),
    )(route_idx, x, expert_W)


def _ring_allreduce(partial):

    def body(in_ref, out_ref, comm_ref, send_sems, recv_sems, copy_sem):
        my = lax.axis_index("i")
        left = lax.rem(my + N_DEV - 1, N_DEV)
        right = lax.rem(my + 1, N_DEV)

        cp = pltpu.make_async_copy(in_ref, out_ref, copy_sem)
        cp.start()
        cp.wait()

        barrier = pltpu.get_barrier_semaphore()
        _sem_signal(
            barrier, inc=1, device_id=(left,), device_id_type=_DevIdType.MESH
        )
        _sem_signal(
            barrier, inc=1, device_id=(right,), device_id_type=_DevIdType.MESH
        )
        _sem_wait(barrier, 2)

        for s in range(N_DEV - 1):
            sc = lax.rem(my - s + N_DEV, N_DEV)
            rc = lax.rem(my - s - 1 + N_DEV, N_DEV)
            rdma = pltpu.make_async_remote_copy(
                src_ref=out_ref.at[pl.ds(sc * CHUNK, CHUNK), :],
                dst_ref=comm_ref.at[s],
                send_sem=send_sems.at[s],
                recv_sem=recv_sems.at[s],
                device_id=(right,),
                device_id_type=_DevIdType.MESH,
            )
            rdma.start()
            rdma.wait()
            out_ref[pl.ds(rc * CHUNK, CHUNK), :] += comm_ref[s]

        for s in range(N_DEV - 1):
            sc = lax.rem(my + 1 - s + N_DEV, N_DEV)
            h = N_DEV - 1 + s
            rdma = pltpu.make_async_remote_copy(
                src_ref=out_ref.at[pl.ds(sc * CHUNK, CHUNK), :],
                dst_ref=out_ref.at[pl.ds(sc * CHUNK, CHUNK), :],
                send_sem=send_sems.at[h],
                recv_sem=recv_sems.at[h],
                device_id=(right,),
                device_id_type=_DevIdType.MESH,
            )
            rdma.start()
            rdma.wait()

    return pl.pallas_call(
        body,
        out_shape=jax.ShapeDtypeStruct((N_TOK, D_OUT), jnp.float32),
        in_specs=[pl.BlockSpec(memory_space=_ANY)],
        out_specs=pl.BlockSpec(memory_space=_VMEM_SPACE),
        scratch_shapes=[
            pltpu.VMEM((N_DEV - 1, CHUNK, D_OUT), jnp.float32),
            pltpu.SemaphoreType.DMA((2 * (N_DEV - 1),)),
            pltpu.SemaphoreType.DMA((2 * (N_DEV - 1),)),
            pltpu.SemaphoreType.DMA,
        ],
        compiler_params=_CompilerParams(
            collective_id=0,
            vmem_limit_bytes=60 * 1024 * 1024,
        ),
    )(partial)


def kernel(x, router_W, route_idx, expert_W):
    del router_W
    partial = _compute_partial(x, route_idx, expert_W)
    return _ring_allreduce(partial)


# device time: 292711 ns/iter; 2.4699x vs baseline; 2.4699x over previous
import jax
import jax.numpy as jnp
from jax import lax
from jax.experimental import pallas as pl
from jax.experimental.pallas import tpu as pltpu

N_DEV = 4
N_TOK = 4096
D_IN = 1024
D_OUT = 2048
HALF = D_OUT // 2
E_LOCAL = 4
TOK_BLK = 512
CHUNK = N_TOK // N_DEV

_DevIdType = getattr(pl, "DeviceIdType", None) or pltpu.DeviceIdType
_CompilerParams = getattr(pltpu, "CompilerParams", None) or getattr(
    pltpu, "TPUCompilerParams"
)
_sem_signal = getattr(pl, "semaphore_signal", None) or pltpu.semaphore_signal
_sem_wait = getattr(pl, "semaphore_wait", None) or pltpu.semaphore_wait
_ANY = pl.ANY
_VMEM_SPACE = pltpu.MemorySpace.VMEM


def _compute_partial(x, route_idx, expert_W):
    n_i = N_TOK // TOK_BLK

    def body(route_ref, x_ref, w_ref, out_ref):
        my = lax.axis_index("i")
        e = pl.program_id(1)
        eid = my * E_LOCAL + e
        mask = route_ref[...] == eid
        xm = jnp.where(mask, x_ref[...], 0.0).astype(jnp.bfloat16)
        acc = jnp.dot(
            xm,
            w_ref[0].astype(jnp.bfloat16),
            preferred_element_type=jnp.float32,
        ).astype(jnp.bfloat16)

        @pl.when(e == 0)
        def _():
            out_ref[...] = acc

        @pl.when(e != 0)
        def _():
            out_ref[...] += acc

    return pl.pallas_call(
        body,
        grid=(n_i, E_LOCAL),
        in_specs=[
            pl.BlockSpec((TOK_BLK, 1), lambda i, e: (i, 0)),
            pl.BlockSpec((TOK_BLK, D_IN), lambda i, e: (i, 0)),
            pl.BlockSpec((1, D_IN, D_OUT), lambda i, e: (e, 0, 0)),
        ],
        out_specs=pl.BlockSpec((TOK_BLK, D_OUT), lambda i, e: (i, 0)),
        out_shape=jax.ShapeDtypeStruct((N_TOK, D_OUT), jnp.bfloat16),
        compiler_params=_CompilerParams(
            dimension_semantics=("arbitrary", "arbitrary"),
        ),
    )(route_idx, x, expert_W)


def _ring_allreduce(partial):

    def body(
        in_ref,
        out_ref,
        comm_ref,
        send_r,
        recv_r,
        send_l,
        recv_l,
        copy_sem,
    ):
        my = lax.axis_index("i")
        left = lax.rem(my + N_DEV - 1, N_DEV)
        right = lax.rem(my + 1, N_DEV)

        cp = pltpu.make_async_copy(in_ref, out_ref, copy_sem)
        cp.start()
        cp.wait()

        barrier = pltpu.get_barrier_semaphore()
        _sem_signal(
            barrier, inc=1, device_id=(left,), device_id_type=_DevIdType.MESH
        )
        _sem_signal(
            barrier, inc=1, device_id=(right,), device_id_type=_DevIdType.MESH
        )
        _sem_wait(barrier, 2)

        for s in range(N_DEV - 1):
            sc_r = lax.rem(my - s + N_DEV, N_DEV)
            rc_r = lax.rem(my - s - 1 + N_DEV, N_DEV)
            sc_l = lax.rem(my + s, N_DEV)
            rc_l = lax.rem(my + s + 1, N_DEV)
            rdma_r = pltpu.make_async_remote_copy(
                src_ref=out_ref.at[pl.ds(sc_r * CHUNK, CHUNK), 0:HALF],
                dst_ref=comm_ref.at[s, :, 0:HALF],
                send_sem=send_r.at[s],
                recv_sem=recv_r.at[s],
                device_id=(right,),
                device_id_type=_DevIdType.MESH,
            )
            rdma_l = pltpu.make_async_remote_copy(
                src_ref=out_ref.at[pl.ds(sc_l * CHUNK, CHUNK), HALF:D_OUT],
                dst_ref=comm_ref.at[s, :, HALF:D_OUT],
                send_sem=send_l.at[s],
                recv_sem=recv_l.at[s],
                device_id=(left,),
                device_id_type=_DevIdType.MESH,
            )
            rdma_r.start()
            rdma_l.start()
            rdma_r.wait()
            rdma_l.wait()
            out_ref[pl.ds(rc_r * CHUNK, CHUNK), 0:HALF] += comm_ref[
                s, :, 0:HALF
            ]
            out_ref[pl.ds(rc_l * CHUNK, CHUNK), HALF:D_OUT] += comm_ref[
                s, :, HALF:D_OUT
            ]

        for s in range(N_DEV - 1):
            h = N_DEV - 1 + s
            c_r = lax.rem(my + 1 - s + N_DEV, N_DEV)
            c_l = lax.rem(my - 1 + s + N_DEV, N_DEV)
            rdma_r = pltpu.make_async_remote_copy(
                src_ref=out_ref.at[pl.ds(c_r * CHUNK, CHUNK), 0:HALF],
                dst_ref=out_ref.at[pl.ds(c_r * CHUNK, CHUNK), 0:HALF],
                send_sem=send_r.at[h],
                recv_sem=recv_r.at[h],
                device_id=(right,),
                device_id_type=_DevIdType.MESH,
            )
            rdma_l = pltpu.make_async_remote_copy(
                src_ref=out_ref.at[pl.ds(c_l * CHUNK, CHUNK), HALF:D_OUT],
                dst_ref=out_ref.at[pl.ds(c_l * CHUNK, CHUNK), HALF:D_OUT],
                send_sem=send_l.at[h],
                recv_sem=recv_l.at[h],
                device_id=(left,),
                device_id_type=_DevIdType.MESH,
            )
            rdma_r.start()
            rdma_l.start()
            rdma_r.wait()
            rdma_l.wait()

    n_hops = 2 * (N_DEV - 1)
    return pl.pallas_call(
        body,
        out_shape=jax.ShapeDtypeStruct((N_TOK, D_OUT), jnp.bfloat16),
        in_specs=[pl.BlockSpec(memory_space=_ANY)],
        out_specs=pl.BlockSpec(memory_space=_VMEM_SPACE),
        scratch_shapes=[
            pltpu.VMEM((N_DEV - 1, CHUNK, D_OUT), jnp.bfloat16),
            pltpu.SemaphoreType.DMA((n_hops,)),
            pltpu.SemaphoreType.DMA((n_hops,)),
            pltpu.SemaphoreType.DMA((n_hops,)),
            pltpu.SemaphoreType.DMA((n_hops,)),
            pltpu.SemaphoreType.DMA,
        ],
        compiler_params=_CompilerParams(
            collective_id=0,
            vmem_limit_bytes=60 * 1024 * 1024,
        ),
    )(partial)


def _cast_f32(y):
    n_i = N_TOK // TOK_BLK

    def body(in_ref, out_ref):
        out_ref[...] = in_ref[...].astype(jnp.float32)

    return pl.pallas_call(
        body,
        grid=(n_i,),
        in_specs=[pl.BlockSpec((TOK_BLK, D_OUT), lambda i: (i, 0))],
        out_specs=pl.BlockSpec((TOK_BLK, D_OUT), lambda i: (i, 0)),
        out_shape=jax.ShapeDtypeStruct((N_TOK, D_OUT), jnp.float32),
    )(y)


def kernel(x, router_W, route_idx, expert_W):
    del router_W
    partial = _compute_partial(x, route_idx, expert_W)
    reduced = _ring_allreduce(partial)
    return _cast_f32(reduced)


# device time: 233355 ns/iter; 3.0981x vs baseline; 1.2544x over previous
import jax
import jax.numpy as jnp
from jax import lax
from jax.experimental import pallas as pl
from jax.experimental.pallas import tpu as pltpu

N_DEV = 4
N_TOK = 4096
D_IN = 1024
D_OUT = 2048
HALF = D_OUT // 2
E_LOCAL = 4
TOK_BLK = 512
CHUNK = N_TOK // N_DEV

_DevIdType = getattr(pl, "DeviceIdType", None) or pltpu.DeviceIdType
_CompilerParams = getattr(pltpu, "CompilerParams", None) or getattr(
    pltpu, "TPUCompilerParams"
)
_sem_signal = getattr(pl, "semaphore_signal", None) or pltpu.semaphore_signal
_sem_wait = getattr(pl, "semaphore_wait", None) or pltpu.semaphore_wait
_ANY = pl.ANY
_VMEM_SPACE = pltpu.MemorySpace.VMEM
_MESH = _DevIdType.MESH


def _cast_w_bf16(expert_W):
    def body(in_ref, out_ref):
        out_ref[...] = in_ref[...].astype(jnp.bfloat16)

    return pl.pallas_call(
        body,
        grid=(E_LOCAL,),
        in_specs=[pl.BlockSpec((1, D_IN, D_OUT), lambda e: (e, 0, 0))],
        out_specs=pl.BlockSpec((1, D_IN, D_OUT), lambda e: (e, 0, 0)),
        out_shape=jax.ShapeDtypeStruct((E_LOCAL, D_IN, D_OUT), jnp.bfloat16),
    )(expert_W)


def _fused_moe_ar(x, route_idx, w_bf16):
    def body(
        route_ref,
        x_hbm,
        w_hbm,
        out_ref,
        xbuf,
        wbuf,
        comm,
        xsems,
        wsems,
        send_r,
        recv_r,
        send_l,
        recv_l,
    ):
        my = lax.axis_index("i")
        left = lax.rem(my + N_DEV - 1, N_DEV)
        right = lax.rem(my + 1, N_DEV)

        offs = [
            lax.rem(my + d + N_DEV, N_DEV) * CHUNK for d in (0, -1, 1, 2)
        ]

        def x_copy(k):
            return pltpu.make_async_copy(
                x_hbm.at[pl.ds(offs[k], CHUNK), :],
                xbuf.at[k % 2],
                xsems.at[k % 2],
            )

        def w_copy(j):
            return pltpu.make_async_copy(
                w_hbm.at[j % E_LOCAL], wbuf.at[j % 2], wsems.at[j % 2]
            )

        def rs_rdma(s, rightward):
            if rightward:
                sc = lax.rem(my - s + N_DEV, N_DEV)
                return pltpu.make_async_remote_copy(
                    src_ref=out_ref.at[pl.ds(sc * CHUNK, CHUNK), 0:HALF],
                    dst_ref=comm.at[s, :, 0:HALF],
                    send_sem=send_r.at[s],
                    recv_sem=recv_r.at[s],
                    device_id=(right,),
                    device_id_type=_MESH,
                )
            sc = lax.rem(my + s, N_DEV)
            return pltpu.make_async_remote_copy(
                src_ref=out_ref.at[pl.ds(sc * CHUNK, CHUNK), HALF:D_OUT],
                dst_ref=comm.at[s, :, HALF:D_OUT],
                send_sem=send_l.at[s],
                recv_sem=recv_l.at[s],
                device_id=(left,),
                device_id_type=_MESH,
            )

        x_copy(0).start()
        w_copy(0).start()

        barrier = pltpu.get_barrier_semaphore()
        _sem_signal(barrier, inc=1, device_id=(left,), device_id_type=_MESH)
        _sem_signal(barrier, inc=1, device_id=(right,), device_id_type=_MESH)
        _sem_wait(barrier, 2)

        def compute_chunk(k):
            x_copy(k).wait()
            if k < N_DEV - 1:
                x_copy(k + 1).start()
            rows = route_ref[pl.ds(offs[k], CHUNK), :]
            for e in range(E_LOCAL):
                j = E_LOCAL * k + e
                w_copy(j).wait()
                if j < E_LOCAL * N_DEV - 1:
                    w_copy(j + 1).start()
                mask = rows == my * E_LOCAL + e
                xm = jnp.where(mask, xbuf[k % 2], 0.0).astype(jnp.bfloat16)
                for h in range(2):
                    cols = slice(h * HALF, (h + 1) * HALF)
                    prod = jnp.dot(
                        xm,
                        wbuf[j % 2][:, cols],
                        preferred_element_type=jnp.float32,
                    ).astype(jnp.bfloat16)
                    if e == 0:
                        out_ref[pl.ds(offs[k], CHUNK), cols] = prod
                    else:
                        out_ref[pl.ds(offs[k], CHUNK), cols] += prod

        compute_chunk(0)
        rs_rdma(0, True).start()
        rs_rdma(0, False).start()

        compute_chunk(1)
        rs_rdma(0, True).wait()
        out_ref[pl.ds(offs[1], CHUNK), 0:HALF] += comm[0, :, 0:HALF]
        rs_rdma(1, True).start()

        compute_chunk(2)
        rs_rdma(0, False).wait()
        out_ref[pl.ds(offs[2], CHUNK), HALF:D_OUT] += comm[0, :, HALF:D_OUT]
        rs_rdma(1, False).start()

        compute_chunk(3)
        rs_rdma(1, True).wait()
        out_ref[pl.ds(offs[3], CHUNK), 0:HALF] += comm[1, :, 0:HALF]
        rs_rdma(2, True).start()
        rs_rdma(1, False).wait()
        out_ref[pl.ds(offs[3], CHUNK), HALF:D_OUT] += comm[1, :, HALF:D_OUT]
        rs_rdma(2, False).start()
        rs_rdma(2, True).wait()
        out_ref[pl.ds(offs[2], CHUNK), 0:HALF] += comm[2, :, 0:HALF]
        rs_rdma(2, False).wait()
        out_ref[pl.ds(offs[1], CHUNK), HALF:D_OUT] += comm[2, :, HALF:D_OUT]

        for s in range(N_DEV - 1):
            h = N_DEV - 1 + s
            c_r = lax.rem(my + 1 - s + N_DEV, N_DEV)
            c_l = lax.rem(my - 1 + s + N_DEV, N_DEV)
            ag_r = pltpu.make_async_remote_copy(
                src_ref=out_ref.at[pl.ds(c_r * CHUNK, CHUNK), 0:HALF],
                dst_ref=out_ref.at[pl.ds(c_r * CHUNK, CHUNK), 0:HALF],
                send_sem=send_r.at[h],
                recv_sem=recv_r.at[h],
                device_id=(right,),
                device_id_type=_MESH,
            )
            ag_l = pltpu.make_async_remote_copy(
                src_ref=out_ref.at[pl.ds(c_l * CHUNK, CHUNK), HALF:D_OUT],
                dst_ref=out_ref.at[pl.ds(c_l * CHUNK, CHUNK), HALF:D_OUT],
                send_sem=send_l.at[h],
                recv_sem=recv_l.at[h],
                device_id=(left,),
                device_id_type=_MESH,
            )
            ag_r.start()
            ag_l.start()
            ag_r.wait()
            ag_l.wait()

    n_hops = 2 * (N_DEV - 1)
    return pl.pallas_call(
        body,
        out_shape=jax.ShapeDtypeStruct((N_TOK, D_OUT), jnp.bfloat16),
        in_specs=[
            pl.BlockSpec(memory_space=_VMEM_SPACE),
            pl.BlockSpec(memory_space=_ANY),
            pl.BlockSpec(memory_space=_ANY),
        ],
        out_specs=pl.BlockSpec(memory_space=_VMEM_SPACE),
        scratch_shapes=[
            pltpu.VMEM((2, CHUNK, D_IN), jnp.float32),
            pltpu.VMEM((2, D_IN, D_OUT), jnp.bfloat16),
            pltpu.VMEM((N_DEV - 1, CHUNK, D_OUT), jnp.bfloat16),
            pltpu.SemaphoreType.DMA((2,)),
            pltpu.SemaphoreType.DMA((2,)),
            pltpu.SemaphoreType.DMA((n_hops,)),
            pltpu.SemaphoreType.DMA((n_hops,)),
            pltpu.SemaphoreType.DMA((n_hops,)),
            pltpu.SemaphoreType.DMA((n_hops,)),
        ],
        compiler_params=_CompilerParams(
            collective_id=0,
            vmem_limit_bytes=63 * 1024 * 1024,
        ),
    )(route_idx, x, w_bf16)


def _cast_f32(y):
    n_i = N_TOK // TOK_BLK

    def body(in_ref, out_ref):
        out_ref[...] = in_ref[...].astype(jnp.float32)

    return pl.pallas_call(
        body,
        grid=(n_i,),
        in_specs=[pl.BlockSpec((TOK_BLK, D_OUT), lambda i: (i, 0))],
        out_specs=pl.BlockSpec((TOK_BLK, D_OUT), lambda i: (i, 0)),
        out_shape=jax.ShapeDtypeStruct((N_TOK, D_OUT), jnp.float32),
    )(y)


def kernel(x, router_W, route_idx, expert_W):
    del router_W
    w_bf16 = _cast_w_bf16(expert_W)
    reduced = _fused_moe_ar(x, route_idx, w_bf16)
    return _cast_f32(reduced)


# device time: 225741 ns/iter; 3.2026x vs baseline; 1.0337x over previous
import jax
import jax.numpy as jnp
from jax import lax
from jax.experimental import pallas as pl
from jax.experimental.pallas import tpu as pltpu

N_DEV = 4
N_TOK = 4096
D_IN = 1024
D_OUT = 2048
HALF = D_OUT // 2
E_LOCAL = 4
TOK_BLK = 512
CHUNK = N_TOK // N_DEV
N_STRIP = 4
HW = HALF // N_STRIP

_DevIdType = getattr(pl, "DeviceIdType", None) or pltpu.DeviceIdType
_CompilerParams = getattr(pltpu, "CompilerParams", None) or getattr(
    pltpu, "TPUCompilerParams"
)
_sem_signal = getattr(pl, "semaphore_signal", None) or pltpu.semaphore_signal
_sem_wait = getattr(pl, "semaphore_wait", None) or pltpu.semaphore_wait
_ANY = pl.ANY
_VMEM_SPACE = pltpu.MemorySpace.VMEM
_MESH = _DevIdType.MESH


def _cast_w_bf16(expert_W):
    def body(in_ref, out_ref):
        out_ref[...] = in_ref[...].astype(jnp.bfloat16)

    return pl.pallas_call(
        body,
        grid=(E_LOCAL,),
        in_specs=[pl.BlockSpec((1, D_IN, D_OUT), lambda e: (e, 0, 0))],
        out_specs=pl.BlockSpec((1, D_IN, D_OUT), lambda e: (e, 0, 0)),
        out_shape=jax.ShapeDtypeStruct((E_LOCAL, D_IN, D_OUT), jnp.bfloat16),
    )(expert_W)


def _fused_moe_ar(x, route_idx, w_bf16):
    def body(
        route_ref,
        x_hbm,
        w_hbm,
        out_ref,
        xbuf,
        wbuf,
        comm,
        xsems,
        wsems,
        send_r,
        recv_r,
        send_l,
        recv_l,
    ):
        my = lax.axis_index("i")
        left = lax.rem(my + N_DEV - 1, N_DEV)
        right = lax.rem(my + 1, N_DEV)

        offs = [
            lax.rem(my + d + N_DEV, N_DEV) * CHUNK for d in (0, -1, 1, 2)
        ]

        def x_copy(k):
            return pltpu.make_async_copy(
                x_hbm.at[pl.ds(offs[k], CHUNK), :],
                xbuf.at[k % 2],
                xsems.at[k % 2],
            )

        def w_copy(j):
            return pltpu.make_async_copy(
                w_hbm.at[j % E_LOCAL], wbuf.at[j % 2], wsems.at[j % 2]
            )

        def colr(t):
            return slice(t * HW, (t + 1) * HW)

        def coll(t):
            return slice(HALF + t * HW, HALF + (t + 1) * HW)

        def rs_rdma(s, rightward, t):
            if rightward:
                sc = lax.rem(my - s + N_DEV, N_DEV)
                cols, dev, ss, rs_ = colr(t), right, send_r, recv_r
            else:
                sc = lax.rem(my + s, N_DEV)
                cols, dev, ss, rs_ = coll(t), left, send_l, recv_l
            return pltpu.make_async_remote_copy(
                src_ref=out_ref.at[pl.ds(sc * CHUNK, CHUNK), cols],
                dst_ref=comm.at[s, :, cols],
                send_sem=ss.at[s, t],
                recv_sem=rs_.at[s, t],
                device_id=(dev,),
                device_id_type=_MESH,
            )

        def ag_rdma(s, rightward, t):
            h = N_DEV - 1 + s
            if rightward:
                c = lax.rem(my + 1 - s + N_DEV, N_DEV)
                cols, dev, ss, rs_ = colr(t), right, send_r, recv_r
            else:
                c = lax.rem(my - 1 + s + N_DEV, N_DEV)
                cols, dev, ss, rs_ = coll(t), left, send_l, recv_l
            return pltpu.make_async_remote_copy(
                src_ref=out_ref.at[pl.ds(c * CHUNK, CHUNK), cols],
                dst_ref=out_ref.at[pl.ds(c * CHUNK, CHUNK), cols],
                send_sem=ss.at[h, t],
                recv_sem=rs_.at[h, t],
                device_id=(dev,),
                device_id_type=_MESH,
            )

        x_copy(0).start()
        w_copy(0).start()

        barrier = pltpu.get_barrier_semaphore()
        _sem_signal(barrier, inc=1, device_id=(left,), device_id_type=_MESH)
        _sem_signal(barrier, inc=1, device_id=(right,), device_id_type=_MESH)
        _sem_wait(barrier, 2)

        def compute_chunk(k):
            x_copy(k).wait()
            if k < N_DEV - 1:
                x_copy(k + 1).start()
            rows = route_ref[pl.ds(offs[k], CHUNK), :]
            for e in range(E_LOCAL):
                j = E_LOCAL * k + e
                w_copy(j).wait()
                if j < E_LOCAL * N_DEV - 1:
                    w_copy(j + 1).start()
                mask = rows == my * E_LOCAL + e
                xm = jnp.where(mask, xbuf[k % 2], 0.0).astype(jnp.bfloat16)
                for h in range(2):
                    cols = slice(h * HALF, (h + 1) * HALF)
                    prod = jnp.dot(
                        xm,
                        wbuf[j % 2][:, cols],
                        preferred_element_type=jnp.float32,
                    ).astype(jnp.bfloat16)
                    if e == 0:
                        out_ref[pl.ds(offs[k], CHUNK), cols] = prod
                    else:
                        out_ref[pl.ds(offs[k], CHUNK), cols] += prod

        compute_chunk(0)
        for t in range(N_STRIP):
            rs_rdma(0, True, t).start()
            rs_rdma(0, False, t).start()

        compute_chunk(1)
        for t in range(N_STRIP):
            rs_rdma(0, True, t).wait()
            out_ref[pl.ds(offs[1], CHUNK), colr(t)] += comm[0, :, colr(t)]
            rs_rdma(1, True, t).start()

        compute_chunk(2)
        for t in range(N_STRIP):
            rs_rdma(0, False, t).wait()
            out_ref[pl.ds(offs[2], CHUNK), coll(t)] += comm[0, :, coll(t)]
            rs_rdma(1, False, t).start()

        compute_chunk(3)
        for t in range(N_STRIP):
            rs_rdma(1, True, t).wait()
            out_ref[pl.ds(offs[3], CHUNK), colr(t)] += comm[1, :, colr(t)]
            rs_rdma(2, True, t).start()
        for t in range(N_STRIP):
            rs_rdma(1, False, t).wait()
            out_ref[pl.ds(offs[3], CHUNK), coll(t)] += comm[1, :, coll(t)]
            rs_rdma(2, False, t).start()
        for t in range(N_STRIP):
            rs_rdma(2, True, t).wait()
            out_ref[pl.ds(offs[2], CHUNK), colr(t)] += comm[2, :, colr(t)]
            ag_rdma(0, True, t).start()
        for t in range(N_STRIP):
            rs_rdma(2, False, t).wait()
            out_ref[pl.ds(offs[1], CHUNK), coll(t)] += comm[2, :, coll(t)]
            ag_rdma(0, False, t).start()

        for s in range(N_DEV - 1):
            for t in range(N_STRIP):
                ag_rdma(s, True, t).wait()
                if s < N_DEV - 2:
                    ag_rdma(s + 1, True, t).start()
            for t in range(N_STRIP):
                ag_rdma(s, False, t).wait()
                if s < N_DEV - 2:
                    ag_rdma(s + 1, False, t).start()

    n_hops = 2 * (N_DEV - 1)
    return pl.pallas_call(
        body,
        out_shape=jax.ShapeDtypeStruct((N_TOK, D_OUT), jnp.bfloat16),
        in_specs=[
            pl.BlockSpec(memory_space=_VMEM_SPACE),
            pl.BlockSpec(memory_space=_ANY),
            pl.BlockSpec(memory_space=_ANY),
        ],
        out_specs=pl.BlockSpec(memory_space=_VMEM_SPACE),
        scratch_shapes=[
            pltpu.VMEM((2, CHUNK, D_IN), jnp.float32),
            pltpu.VMEM((2, D_IN, D_OUT), jnp.bfloat16),
            pltpu.VMEM((N_DEV - 1, CHUNK, D_OUT), jnp.bfloat16),
            pltpu.SemaphoreType.DMA((2,)),
            pltpu.SemaphoreType.DMA((2,)),
            pltpu.SemaphoreType.DMA((n_hops, N_STRIP)),
            pltpu.SemaphoreType.DMA((n_hops, N_STRIP)),
            pltpu.SemaphoreType.DMA((n_hops, N_STRIP)),
            pltpu.SemaphoreType.DMA((n_hops, N_STRIP)),
        ],
        compiler_params=_CompilerParams(
            collective_id=0,
            vmem_limit_bytes=63 * 1024 * 1024,
        ),
    )(route_idx, x, w_bf16)


def _cast_f32(y):
    n_i = N_TOK // TOK_BLK

    def body(in_ref, out_ref):
        out_ref[...] = in_ref[...].astype(jnp.float32)

    return pl.pallas_call(
        body,
        grid=(n_i,),
        in_specs=[pl.BlockSpec((TOK_BLK, D_OUT), lambda i: (i, 0))],
        out_specs=pl.BlockSpec((TOK_BLK, D_OUT), lambda i: (i, 0)),
        out_shape=jax.ShapeDtypeStruct((N_TOK, D_OUT), jnp.float32),
    )(y)


def kernel(x, router_W, route_idx, expert_W):
    del router_W
    w_bf16 = _cast_w_bf16(expert_W)
    reduced = _fused_moe_ar(x, route_idx, w_bf16)
    return _cast_f32(reduced)


# device time: 225652 ns/iter; 3.2039x vs baseline; 1.0004x over previous
import jax
import jax.numpy as jnp
from jax import lax
from jax.experimental import pallas as pl
from jax.experimental.pallas import tpu as pltpu

N_DEV = 4
N_TOK = 4096
D_IN = 1024
D_OUT = 2048
HALF = D_OUT // 2
E_LOCAL = 4
TOK_BLK = 512
CHUNK = N_TOK // N_DEV
N_STRIP = 4
RS_H = CHUNK // N_STRIP

_DevIdType = getattr(pl, "DeviceIdType", None) or pltpu.DeviceIdType
_CompilerParams = getattr(pltpu, "CompilerParams", None) or getattr(
    pltpu, "TPUCompilerParams"
)
_sem_signal = getattr(pl, "semaphore_signal", None) or pltpu.semaphore_signal
_sem_wait = getattr(pl, "semaphore_wait", None) or pltpu.semaphore_wait
_ANY = pl.ANY
_VMEM_SPACE = pltpu.MemorySpace.VMEM
_MESH = _DevIdType.MESH


def _cast_w_bf16(expert_W):
    def body(in_ref, out_ref):
        out_ref[...] = in_ref[...].astype(jnp.bfloat16)

    return pl.pallas_call(
        body,
        grid=(E_LOCAL,),
        in_specs=[pl.BlockSpec((1, D_IN, D_OUT), lambda e: (e, 0, 0))],
        out_specs=pl.BlockSpec((1, D_IN, D_OUT), lambda e: (e, 0, 0)),
        out_shape=jax.ShapeDtypeStruct((E_LOCAL, D_IN, D_OUT), jnp.bfloat16),
    )(expert_W)


def _fused_moe_ar(x, route_idx, w_bf16):
    def body(
        route_ref,
        x_hbm,
        w_hbm,
        out_ref,
        xbuf,
        wbuf,
        comm,
        xsems,
        wsems,
        send_r,
        recv_r,
        send_l,
        recv_l,
    ):
        my = lax.axis_index("i")
        left = lax.rem(my + N_DEV - 1, N_DEV)
        right = lax.rem(my + 1, N_DEV)

        offs = [
            lax.rem(my + d + N_DEV, N_DEV) * CHUNK for d in (0, -1, 1, 2)
        ]

        def x_copy(k):
            return pltpu.make_async_copy(
                x_hbm.at[pl.ds(offs[k], CHUNK), :],
                xbuf.at[k % 2],
                xsems.at[k % 2],
            )

        def w_copy(j):
            return pltpu.make_async_copy(
                w_hbm.at[j % E_LOCAL], wbuf.at[j % 2], wsems.at[j % 2]
            )

        _COLR = slice(0, HALF)
        _COLL = slice(HALF, D_OUT)

        def rs_rdma(s, rightward, t):
            if rightward:
                sc = lax.rem(my - s + N_DEV, N_DEV)
                cols, dev, ss, rs_ = _COLR, right, send_r, recv_r
            else:
                sc = lax.rem(my + s, N_DEV)
                cols, dev, ss, rs_ = _COLL, left, send_l, recv_l
            return pltpu.make_async_remote_copy(
                src_ref=out_ref.at[
                    pl.ds(sc * CHUNK + t * RS_H, RS_H), cols
                ],
                dst_ref=comm.at[s, pl.ds(t * RS_H, RS_H), cols],
                send_sem=ss.at[s, t],
                recv_sem=rs_.at[s, t],
                device_id=(dev,),
                device_id_type=_MESH,
            )

        def ag_rdma(s, rightward, t):
            h = N_DEV - 1 + s
            if rightward:
                c = lax.rem(my + 1 - s + N_DEV, N_DEV)
                cols, dev, ss, rs_ = _COLR, right, send_r, recv_r
            else:
                c = lax.rem(my - 1 + s + N_DEV, N_DEV)
                cols, dev, ss, rs_ = _COLL, left, send_l, recv_l
            rows = pl.ds(c * CHUNK + t * RS_H, RS_H)
            return pltpu.make_async_remote_copy(
                src_ref=out_ref.at[rows, cols],
                dst_ref=out_ref.at[rows, cols],
                send_sem=ss.at[h, t],
                recv_sem=rs_.at[h, t],
                device_id=(dev,),
                device_id_type=_MESH,
            )

        x_copy(0).start()
        w_copy(0).start()

        barrier = pltpu.get_barrier_semaphore()
        _sem_signal(barrier, inc=1, device_id=(left,), device_id_type=_MESH)
        _sem_signal(barrier, inc=1, device_id=(right,), device_id_type=_MESH)
        _sem_wait(barrier, 2)

        def compute_chunk(k):
            x_copy(k).wait()
            if k < N_DEV - 1:
                x_copy(k + 1).start()
            rows = route_ref[pl.ds(offs[k], CHUNK), :]
            for e in range(E_LOCAL):
                j = E_LOCAL * k + e
                w_copy(j).wait()
                if j < E_LOCAL * N_DEV - 1:
                    w_copy(j + 1).start()
                mask = rows == my * E_LOCAL + e
                xm = jnp.where(mask, xbuf[k % 2], 0.0).astype(jnp.bfloat16)
                for h in range(2):
                    cols = slice(h * HALF, (h + 1) * HALF)
                    prod = jnp.dot(
                        xm,
                        wbuf[j % 2][:, cols],
                        preferred_element_type=jnp.float32,
                    ).astype(jnp.bfloat16)
                    if e == 0:
                        out_ref[pl.ds(offs[k], CHUNK), cols] = prod
                    else:
                        out_ref[pl.ds(offs[k], CHUNK), cols] += prod

        compute_chunk(0)
        for t in range(N_STRIP):
            rs_rdma(0, True, t).start()
            rs_rdma(0, False, t).start()

        def rs_add(s, k, cols, t):
            rows = pl.ds(offs[k] + t * RS_H, RS_H)
            crows = slice(t * RS_H, (t + 1) * RS_H)
            out_ref[rows, cols] += comm[s, crows, cols]

        compute_chunk(1)
        for t in range(N_STRIP):
            rs_rdma(0, True, t).wait()
            rs_add(0, 1, _COLR, t)
            rs_rdma(1, True, t).start()

        compute_chunk(2)
        for t in range(N_STRIP):
            rs_rdma(0, False, t).wait()
            rs_add(0, 2, _COLL, t)
            rs_rdma(1, False, t).start()

        compute_chunk(3)
        for t in range(N_STRIP):
            rs_rdma(1, True, t).wait()
            rs_add(1, 3, _COLR, t)
            rs_rdma(2, True, t).start()
        for t in range(N_STRIP):
            rs_rdma(1, False, t).wait()
            rs_add(1, 3, _COLL, t)
            rs_rdma(2, False, t).start()
        for t in range(N_STRIP):
            rs_rdma(2, True, t).wait()
            rs_add(2, 2, _COLR, t)
            ag_rdma(0, True, t).start()
        for t in range(N_STRIP):
            rs_rdma(2, False, t).wait()
            rs_add(2, 1, _COLL, t)
            ag_rdma(0, False, t).start()

        for s in range(N_DEV - 1):
            for t in range(N_STRIP):
                ag_rdma(s, True, t).wait()
                if s < N_DEV - 2:
                    ag_rdma(s + 1, True, t).start()
            for t in range(N_STRIP):
                ag_rdma(s, False, t).wait()
                if s < N_DEV - 2:
                    ag_rdma(s + 1, False, t).start()

    n_hops = 2 * (N_DEV - 1)
    return pl.pallas_call(
        body,
        out_shape=jax.ShapeDtypeStruct((N_TOK, D_OUT), jnp.bfloat16),
        in_specs=[
            pl.BlockSpec(memory_space=_VMEM_SPACE),
            pl.BlockSpec(memory_space=_ANY),
            pl.BlockSpec(memory_space=_ANY),
        ],
        out_specs=pl.BlockSpec(memory_space=_VMEM_SPACE),
        scratch_shapes=[
            pltpu.VMEM((2, CHUNK, D_IN), jnp.float32),
            pltpu.VMEM((2, D_IN, D_OUT), jnp.bfloat16),
            pltpu.VMEM((N_DEV - 1, CHUNK, D_OUT), jnp.bfloat16),
            pltpu.SemaphoreType.DMA((2,)),
            pltpu.SemaphoreType.DMA((2,)),
            pltpu.SemaphoreType.DMA((n_hops, N_STRIP)),
            pltpu.SemaphoreType.DMA((n_hops, N_STRIP)),
            pltpu.SemaphoreType.DMA((n_hops, N_STRIP)),
            pltpu.SemaphoreType.DMA((n_hops, N_STRIP)),
        ],
        compiler_params=_CompilerParams(
            collective_id=0,
            vmem_limit_bytes=63 * 1024 * 1024,
        ),
    )(route_idx, x, w_bf16)


def _cast_f32(y):
    n_i = N_TOK // TOK_BLK

    def body(in_ref, out_ref):
        out_ref[...] = in_ref[...].astype(jnp.float32)

    return pl.pallas_call(
        body,
        grid=(n_i,),
        in_specs=[pl.BlockSpec((TOK_BLK, D_OUT), lambda i: (i, 0))],
        out_specs=pl.BlockSpec((TOK_BLK, D_OUT), lambda i: (i, 0)),
        out_shape=jax.ShapeDtypeStruct((N_TOK, D_OUT), jnp.float32),
    )(y)


def kernel(x, router_W, route_idx, expert_W):
    del router_W
    w_bf16 = _cast_w_bf16(expert_W)
    reduced = _fused_moe_ar(x, route_idx, w_bf16)
    return _cast_f32(reduced)


# device time: 223697 ns/iter; 3.2319x vs baseline; 1.0087x over previous
import jax
import jax.numpy as jnp
from jax import lax
from jax.experimental import pallas as pl
from jax.experimental.pallas import tpu as pltpu

N_DEV = 4
N_TOK = 4096
D_IN = 1024
D_OUT = 2048
HALF = D_OUT // 2
E_LOCAL = 4
CHUNK = N_TOK // N_DEV
N_STRIP = 4
RS_H = CHUNK // N_STRIP

W_SEQ = [0, 1, 2, 3] * (N_DEV + 1)

_DevIdType = getattr(pl, "DeviceIdType", None) or pltpu.DeviceIdType
_CompilerParams = getattr(pltpu, "CompilerParams", None) or getattr(
    pltpu, "TPUCompilerParams"
)
_sem_signal = getattr(pl, "semaphore_signal", None) or pltpu.semaphore_signal
_sem_wait = getattr(pl, "semaphore_wait", None) or pltpu.semaphore_wait
_ANY = pl.ANY
_VMEM_SPACE = pltpu.MemorySpace.VMEM
_MESH = _DevIdType.MESH


def _cast_w_bf16(expert_W):
    def body(in_ref, out_ref):
        out_ref[...] = in_ref[...].astype(jnp.bfloat16)

    return pl.pallas_call(
        body,
        grid=(E_LOCAL,),
        in_specs=[pl.BlockSpec((1, D_IN, D_OUT), lambda e: (e, 0, 0))],
        out_specs=pl.BlockSpec((1, D_IN, D_OUT), lambda e: (e, 0, 0)),
        out_shape=jax.ShapeDtypeStruct((E_LOCAL, D_IN, D_OUT), jnp.bfloat16),
    )(expert_W)


def _fused_moe_ar(x, route_idx, w_bf16):
    def body(
        route_ref,
        x_hbm,
        w_hbm,
        out_hbm,
        xbuf,
        wbuf,
        comm,
        work,
        fstage,
        xsems,
        wsems,
        fsems,
        send_r,
        recv_r,
        send_l,
        recv_l,
    ):
        my = lax.axis_index("i")
        left = lax.rem(my + N_DEV - 1, N_DEV)
        right = lax.rem(my + 1, N_DEV)

        offs = [
            lax.rem(my + d + N_DEV, N_DEV) * CHUNK for d in (0, -1, 1, 2)
        ]

        _COLR = slice(0, HALF)
        _COLL = slice(HALF, D_OUT)

        def x_copy(k):
            return pltpu.make_async_copy(
                x_hbm.at[pl.ds(offs[k], CHUNK), :],
                xbuf.at[k % 2],
                xsems.at[k % 2],
            )

        def w_copy(seq):
            return pltpu.make_async_copy(
                w_hbm.at[W_SEQ[seq]], wbuf.at[seq % 2], wsems.at[seq % 2]
            )

        def rs_rdma(s, rightward, t):
            if rightward:
                sc = lax.rem(my - s + N_DEV, N_DEV)
                cols, dev, ss, rs_ = _COLR, right, send_r, recv_r
            else:
                sc = lax.rem(my + s, N_DEV)
                cols, dev, ss, rs_ = _COLL, left, send_l, recv_l
            return pltpu.make_async_remote_copy(
                src_ref=work.at[pl.ds(sc * CHUNK + t * RS_H, RS_H), cols],
                dst_ref=comm.at[s, pl.ds(t * RS_H, RS_H), cols],
                send_sem=ss.at[s, t],
                recv_sem=rs_.at[s, t],
                device_id=(dev,),
                device_id_type=_MESH,
            )

        def ag_rdma(s, rightward, t):
            h = N_DEV - 1 + s
            if rightward:
                c = lax.rem(my + 1 - s + N_DEV, N_DEV)
                cols, dev, ss, rs_ = _COLR, right, send_r, recv_r
            else:
                c = lax.rem(my - 1 + s + N_DEV, N_DEV)
                cols, dev, ss, rs_ = _COLL, left, send_l, recv_l
            rows = pl.ds(c * CHUNK + t * RS_H, RS_H)
            return pltpu.make_async_remote_copy(
                src_ref=work.at[rows, cols],
                dst_ref=work.at[rows, cols],
                send_sem=ss.at[h, t],
                recv_sem=rs_.at[h, t],
                device_id=(dev,),
                device_id_type=_MESH,
            )

        def rs_add(s, k, cols, t):
            rows = pl.ds(offs[k] + t * RS_H, RS_H)
            crows = slice(t * RS_H, (t + 1) * RS_H)
            work[rows, cols] += comm[s, crows, cols]

        pending = [None, None]
        emit_n = [0]

        def emit(row_start, cols):
            slot = emit_n[0] % 2
            if pending[slot] is not None:
                pending[slot].wait()
            fstage[slot] = work[pl.ds(row_start, RS_H), cols].astype(
                jnp.float32
            )
            cp = pltpu.make_async_copy(
                fstage.at[slot],
                out_hbm.at[pl.ds(row_start, RS_H), cols],
                fsems.at[slot],
            )
            cp.start()
            pending[slot] = cp
            emit_n[0] += 1

        x_copy(0).start()
        w_copy(0).start()

        barrier = pltpu.get_barrier_semaphore()
        _sem_signal(barrier, inc=1, device_id=(left,), device_id_type=_MESH)
        _sem_signal(barrier, inc=1, device_id=(right,), device_id_type=_MESH)
        _sem_wait(barrier, 2)

        def compute_rows(k, r0, r1, seq0):
            rows = route_ref[pl.ds(offs[k] + r0, r1 - r0), :]
            orows = pl.ds(offs[k] + r0, r1 - r0)
            for i in range(E_LOCAL):
                seq = seq0 + i
                w_copy(seq).wait()
                if seq + 1 < len(W_SEQ):
                    w_copy(seq + 1).start()
                mask = rows == my * E_LOCAL + i
                xm = jnp.where(mask, xbuf[k % 2][r0:r1, :], 0.0).astype(
                    jnp.bfloat16
                )
                for h in range(2):
                    cols = slice(h * HALF, (h + 1) * HALF)
                    prod = jnp.dot(
                        xm,
                        wbuf[seq % 2][:, cols],
                        preferred_element_type=jnp.float32,
                    ).astype(jnp.bfloat16)
                    if i == 0:
                        work[orows, cols] = prod
                    else:
                        work[orows, cols] += prod

        x_copy(0).wait()
        x_copy(1).start()
        compute_rows(0, 0, CHUNK // 2, 0)
        for t in range(N_STRIP // 2):
            rs_rdma(0, True, t).start()
            rs_rdma(0, False, t).start()
        compute_rows(0, CHUNK // 2, CHUNK, E_LOCAL)
        for t in range(N_STRIP // 2, N_STRIP):
            rs_rdma(0, True, t).start()
            rs_rdma(0, False, t).start()

        def compute_chunk(k):
            x_copy(k).wait()
            if k < N_DEV - 1:
                x_copy(k + 1).start()
            compute_rows(k, 0, CHUNK, E_LOCAL * (k + 1))

        compute_chunk(1)
        for t in range(N_STRIP):
            rs_rdma(0, True, t).wait()
            rs_add(0, 1, _COLR, t)
            rs_rdma(1, True, t).start()

        compute_chunk(2)
        for t in range(N_STRIP):
            rs_rdma(0, False, t).wait()
            rs_add(0, 2, _COLL, t)
            rs_rdma(1, False, t).start()

        compute_chunk(3)
        for t in range(N_STRIP):
            rs_rdma(1, True, t).wait()
            rs_add(1, 3, _COLR, t)
            rs_rdma(2, True, t).start()
        for t in range(N_STRIP):
            rs_rdma(1, False, t).wait()
            rs_add(1, 3, _COLL, t)
            rs_rdma(2, False, t).start()
        for t in range(N_STRIP):
            rs_rdma(2, True, t).wait()
            rs_add(2, 2, _COLR, t)
            ag_rdma(0, True, t).start()
            emit(offs[2] + t * RS_H, _COLR)
        for t in range(N_STRIP):
            rs_rdma(2, False, t).wait()
            rs_add(2, 1, _COLL, t)
            ag_rdma(0, False, t).start()
            emit(offs[1] + t * RS_H, _COLL)

        for s in range(N_DEV - 1):
            cr = lax.rem(my - s + N_DEV, N_DEV) * CHUNK
            cl = lax.rem(my + s, N_DEV) * CHUNK
            for t in range(N_STRIP):
                ag_rdma(s, True, t).wait()
                if s < N_DEV - 2:
                    ag_rdma(s + 1, True, t).start()
                emit(cr + t * RS_H, _COLR)
            for t in range(N_STRIP):
                ag_rdma(s, False, t).wait()
                if s < N_DEV - 2:
                    ag_rdma(s + 1, False, t).start()
                emit(cl + t * RS_H, _COLL)

        for slot in range(2):
            if pending[slot] is not None:
                pending[slot].wait()

    n_hops = 2 * (N_DEV - 1)
    return pl.pallas_call(
        body,
        out_shape=jax.ShapeDtypeStruct((N_TOK, D_OUT), jnp.float32),
        in_specs=[
            pl.BlockSpec(memory_space=_VMEM_SPACE),
            pl.BlockSpec(memory_space=_ANY),
            pl.BlockSpec(memory_space=_ANY),
        ],
        out_specs=pl.BlockSpec(memory_space=_ANY),
        scratch_shapes=[
            pltpu.VMEM((2, CHUNK, D_IN), jnp.float32),
            pltpu.VMEM((2, D_IN, D_OUT), jnp.bfloat16),
            pltpu.VMEM((N_DEV - 1, CHUNK, D_OUT), jnp.bfloat16),
            pltpu.VMEM((N_TOK, D_OUT), jnp.bfloat16),
            pltpu.VMEM((2, RS_H, HALF), jnp.float32),
            pltpu.SemaphoreType.DMA((2,)),
            pltpu.SemaphoreType.DMA((2,)),
            pltpu.SemaphoreType.DMA((2,)),
            pltpu.SemaphoreType.DMA((n_hops, N_STRIP)),
            pltpu.SemaphoreType.DMA((n_hops, N_STRIP)),
            pltpu.SemaphoreType.DMA((n_hops, N_STRIP)),
            pltpu.SemaphoreType.DMA((n_hops, N_STRIP)),
        ],
        compiler_params=_CompilerParams(
            collective_id=0,
            vmem_limit_bytes=63 * 1024 * 1024,
        ),
    )(route_idx, x, w_bf16)


def kernel(x, router_W, route_idx, expert_W):
    del router_W
    w_bf16 = _cast_w_bf16(expert_W)
    return _fused_moe_ar(x, route_idx, w_bf16)
